# baseline (speedup 1.0000x reference)
import jax
import jax.numpy as jnp
import numpy as np
from jax import lax
from jax.experimental import pallas as pl
from jax.experimental.pallas import tpu as pltpu
from jax.experimental.pallas import tpu_sc as plsc

NC = 2
NS = 16
NW = NC * NS
CHUNK = 128
CPG = 2
GROUP = CHUNK * CPG
L = 16


def _gather_body(table_hbm, idx_hbm, out_hbm, idx_v,
                 p0, p1, f0, f1, g0, g1, w0, w1):
    packed = (p0, p1)
    fbuf = (f0, f1)
    gsem = (g0, g1)
    wsem = (w0, w1)

    n_chunk_rows = idx_hbm.shape[0]
    chunks_per_w = n_chunk_rows // NW
    ngroups = chunks_per_w // CPG
    wid = lax.axis_index("s") * NC + lax.axis_index("c")
    base_chunk = wid * chunks_per_w

    pltpu.sync_copy(idx_hbm.at[pl.ds(base_chunk, chunks_per_w)], idx_v)

    def fire_gathers(g, b):
        for h in range(CPG):
            pltpu.async_copy(
                table_hbm.at[idx_v.at[g * CPG + h]],
                packed[b].at[pl.ds(h * CHUNK, CHUNK)], gsem[b])

    def drain_gathers(g, b):
        for h in range(CPG):
            pltpu.make_async_copy(
                table_hbm.at[idx_v.at[g * CPG + h]],
                packed[b].at[pl.ds(h * CHUNK, CHUNK)], gsem[b]).wait()

    def fire_write(g, b):
        pltpu.async_copy(
            fbuf[b],
            out_hbm.at[pl.ds((base_chunk + g * CPG) * CHUNK, GROUP)],
            wsem[b])

    def drain_write(b):
        pltpu.make_async_copy(
            fbuf[b], out_hbm.at[pl.ds(0, GROUP)], wsem[b]).wait()

    def convert(b):
        src, dst = packed[b], fbuf[b]
        shift16 = jnp.full((L,), 16, jnp.int32)
        hi_mask = jnp.full((L,), -65536, jnp.int32)

        @pl.loop(0, GROUP, unroll=4)
        def _(r):
            for h in range(64 // L):
                v = src[r, pl.ds(h * L, L)]
                dst[r, pl.ds(h * L, L)] = lax.shift_left(v, shift16)
                dst[r, pl.ds(64 + h * L, L)] = lax.bitwise_and(v, hi_mask)

    fire_gathers(0, 0)
    fire_gathers(1, 1)
    drain_gathers(0, 0)
    convert(0)
    fire_write(0, 0)
    fire_gathers(2, 0)
    drain_gathers(1, 1)
    convert(1)
    fire_write(1, 1)

    @pl.loop(2, ngroups - 2, step=2)
    def _(gg):
        for h in range(2):
            g = gg + h
            b = h
            fire_gathers(g + 1, 1 - b)
            drain_gathers(g, b)
            drain_write(b)
            convert(b)
            fire_write(g, b)

    glast = ngroups - 1
    fire_gathers(glast, 1)
    drain_gathers(glast - 1, 0)
    drain_write(0)
    convert(0)
    fire_write(glast - 1, 0)
    drain_gathers(glast, 1)
    drain_write(1)
    convert(1)
    fire_write(glast, 1)
    drain_write(0)
    drain_write(1)


def kernel(x, table):
    B, S = x.shape
    V, D = table.shape
    total = B * S
    idx2d = x.reshape(total // CHUNK, CHUNK)

    perm = np.empty((D,), dtype=np.int32)
    perm[0::2] = np.arange(D // 2)
    perm[1::2] = np.arange(D // 2) + D // 2
    tb = table.astype(jnp.bfloat16)[:, perm]
    ti = lax.bitcast_convert_type(tb.reshape(V, D // 2, 2), jnp.int32)

    mesh = plsc.VectorSubcoreMesh(
        core_axis_name="c", subcore_axis_name="s",
        num_cores=NC, num_subcores=NS)

    run = pl.kernel(
        _gather_body,
        out_type=jax.ShapeDtypeStruct((total, D), jnp.int32),
        mesh=mesh,
        scratch_types=[
            pltpu.VMEM((total // CHUNK // NW, CHUNK), jnp.int32),
            pltpu.VMEM((GROUP, D // 2), jnp.int32),
            pltpu.VMEM((GROUP, D // 2), jnp.int32),
            pltpu.VMEM((GROUP, D), jnp.int32),
            pltpu.VMEM((GROUP, D), jnp.int32),
        ] + [pltpu.SemaphoreType.DMA for _ in range(4)],
        compiler_params=pltpu.CompilerParams(use_tc_tiling_on_sc=False),
    )
    out = run(ti, idx2d)
    return lax.bitcast_convert_type(out, jnp.float32).reshape(B, S, D)


# speedup vs baseline: 1.4049x; 1.4049x over previous
import jax
import jax.numpy as jnp
import numpy as np
from jax import lax
from jax.experimental import pallas as pl
from jax.experimental.pallas import tpu as pltpu
from jax.experimental.pallas import tpu_sc as plsc

NC = 2
NS = 16
NW = NC * NS
CHUNK = 128
CPG = 2
GROUP = CHUNK * CPG
L = 16


def _gather_body(table_hbm, idx_hbm, out_hbm, idx_v,
                 p0, p1, f0, f1, g0, g1, w0, w1):
    packed = (p0, p1)
    fbuf = (f0, f1)
    gsem = (g0, g1)
    wsem = (w0, w1)

    n_chunk_rows = idx_hbm.shape[0]
    chunks_per_w = n_chunk_rows // NW
    ngroups = chunks_per_w // CPG
    wid = lax.axis_index("s") * NC + lax.axis_index("c")
    base_chunk = wid * chunks_per_w

    pltpu.sync_copy(idx_hbm.at[pl.ds(base_chunk, chunks_per_w)], idx_v)

    def fire_gathers(g, b):
        for h in range(CPG):
            pltpu.async_copy(
                table_hbm.at[idx_v.at[g * CPG + h]],
                packed[b].at[pl.ds(h * CHUNK, CHUNK)], gsem[b])

    def drain_gathers(g, b):
        for h in range(CPG):
            pltpu.make_async_copy(
                table_hbm.at[idx_v.at[g * CPG + h]],
                packed[b].at[pl.ds(h * CHUNK, CHUNK)], gsem[b]).wait()

    def fire_write(g, b):
        pltpu.async_copy(
            fbuf[b],
            out_hbm.at[pl.ds((base_chunk + g * CPG) * CHUNK, GROUP)],
            wsem[b])

    def drain_write(b):
        pltpu.make_async_copy(
            fbuf[b], out_hbm.at[pl.ds(0, GROUP)], wsem[b]).wait()

    def convert(b):
        src, dst = packed[b], fbuf[b]
        shift16 = jnp.full((L,), 16, jnp.int32)
        hi_mask = jnp.full((L,), -65536, jnp.int32)

        @pl.loop(0, GROUP, unroll=4)
        def _(r):
            for h in range(64 // L):
                v = src[r, pl.ds(h * L, L)]
                dst[r, pl.ds(h * L, L)] = lax.shift_left(v, shift16)
                dst[r, pl.ds(64 + h * L, L)] = lax.bitwise_and(v, hi_mask)

    fire_gathers(0, 0)
    fire_gathers(1, 1)
    drain_gathers(0, 0)
    convert(0)
    fire_write(0, 0)
    fire_gathers(2, 0)
    drain_gathers(1, 1)
    convert(1)
    fire_write(1, 1)

    @pl.loop(2, ngroups - 2, step=2)
    def _(gg):
        for h in range(2):
            g = gg + h
            b = h
            fire_gathers(g + 1, 1 - b)
            drain_gathers(g, b)
            drain_write(b)
            convert(b)
            fire_write(g, b)

    glast = ngroups - 1
    fire_gathers(glast, 1)
    drain_gathers(glast - 1, 0)
    drain_write(0)
    convert(0)
    fire_write(glast - 1, 0)
    drain_gathers(glast, 1)
    drain_write(1)
    convert(1)
    fire_write(glast, 1)
    drain_write(0)
    drain_write(1)


def kernel(x, table):
    B, S = x.shape
    V, D = table.shape
    total = B * S
    idx2d = x.reshape(total // CHUNK, CHUNK)

    tb = table.astype(jnp.bfloat16).reshape(V, 2, D // 2)
    ti = lax.bitcast_convert_type(
        jnp.stack((tb[:, 0, :], tb[:, 1, :]), axis=-1), jnp.int32)

    mesh = plsc.VectorSubcoreMesh(
        core_axis_name="c", subcore_axis_name="s",
        num_cores=NC, num_subcores=NS)

    run = pl.kernel(
        _gather_body,
        out_type=jax.ShapeDtypeStruct((total, D), jnp.int32),
        mesh=mesh,
        scratch_types=[
            pltpu.VMEM((total // CHUNK // NW, CHUNK), jnp.int32),
            pltpu.VMEM((GROUP, D // 2), jnp.int32),
            pltpu.VMEM((GROUP, D // 2), jnp.int32),
            pltpu.VMEM((GROUP, D), jnp.int32),
            pltpu.VMEM((GROUP, D), jnp.int32),
        ] + [pltpu.SemaphoreType.DMA for _ in range(4)],
        compiler_params=pltpu.CompilerParams(use_tc_tiling_on_sc=False),
    )
    out = run(ti, idx2d)
    return lax.bitcast_convert_type(out, jnp.float32).reshape(B, S, D)


# speedup vs baseline: 3.9956x; 2.8440x over previous
import jax
import jax.numpy as jnp
from jax import lax
from jax.experimental import pallas as pl
from jax.experimental.pallas import tpu as pltpu
from jax.experimental.pallas import tpu_sc as plsc

NC = 2
NS = 16
NW = NC * NS
CHUNK = 128
CPG = 2
GPW_ROWS = CHUNK * CPG


def _gather_body(table_hbm, idx_hbm, out_hbm, idx_v,
                 rows0, rows1, g0, g1, w0, w1):
    rows = (rows0, rows1)
    gsem = (g0, g1)
    wsem = (w0, w1)

    n_chunk_rows = idx_hbm.shape[0]
    chunks_per_w = n_chunk_rows // NW
    ngroups = chunks_per_w // CPG
    wid = lax.axis_index("s") * NC + lax.axis_index("c")
    base_chunk = wid * chunks_per_w

    pltpu.sync_copy(idx_hbm.at[pl.ds(base_chunk, chunks_per_w)], idx_v)

    def fire_gathers(g, b):
        for h in range(CPG):
            pltpu.async_copy(
                table_hbm.at[idx_v.at[g * CPG + h]],
                rows[b].at[pl.ds(h * CHUNK, CHUNK)], gsem[b])

    def drain_gathers(g, b):
        for h in range(CPG):
            pltpu.make_async_copy(
                table_hbm.at[idx_v.at[g * CPG + h]],
                rows[b].at[pl.ds(h * CHUNK, CHUNK)], gsem[b]).wait()

    def fire_write(g, b):
        pltpu.async_copy(
            rows[b],
            out_hbm.at[pl.ds((base_chunk + g * CPG) * CHUNK, GPW_ROWS)],
            wsem[b])

    def drain_write(b):
        pltpu.make_async_copy(
            rows[b], out_hbm.at[pl.ds(0, GPW_ROWS)], wsem[b]).wait()

    fire_gathers(0, 0)
    drain_gathers(0, 0)
    fire_write(0, 0)
    fire_gathers(1, 1)

    @pl.loop(2, ngroups, step=2)
    def _(g):
        for h in range(2):
            b_prev = (h + 1) % 2
            b_cur = h
            drain_gathers(g + h - 1, b_prev)
            fire_write(g + h - 1, b_prev)
            drain_write(b_cur)
            fire_gathers(g + h, b_cur)

    drain_gathers(ngroups - 1, (ngroups - 1) % 2)
    fire_write(ngroups - 1, (ngroups - 1) % 2)
    drain_write(0)
    drain_write(1)


def kernel(x, table):
    B, S = x.shape
    V, D = table.shape
    total = B * S
    idx2d = x.reshape(total // CHUNK, CHUNK)

    mesh = plsc.VectorSubcoreMesh(
        core_axis_name="c", subcore_axis_name="s",
        num_cores=NC, num_subcores=NS)

    run = pl.kernel(
        _gather_body,
        out_type=jax.ShapeDtypeStruct((total, D), jnp.float32),
        mesh=mesh,
        scratch_types=[
            pltpu.VMEM((total // CHUNK // NW, CHUNK), jnp.int32),
            pltpu.VMEM((GPW_ROWS, D), jnp.float32),
            pltpu.VMEM((GPW_ROWS, D), jnp.float32),
            pltpu.SemaphoreType.DMA,
            pltpu.SemaphoreType.DMA,
            pltpu.SemaphoreType.DMA,
            pltpu.SemaphoreType.DMA,
        ],
    )
    out = run(table, idx2d)
    return out.reshape(B, S, D)
